# Initial kernel scaffold; baseline (speedup 1.0000x reference)
import jax
import jax.numpy as jnp
from jax.experimental import pallas as pl


def kernel(x, Wr, W1, W2, W3):
    raise NotImplementedError("write your pallas kernel here")


# speedup vs baseline: 1.5406x; 1.5406x over previous
import functools

import jax
import jax.numpy as jnp
from jax import lax
from jax.experimental import pallas as pl
from jax.experimental.pallas import tpu as pltpu
from jax.experimental.pallas import tpu_sc as plsc

_TOP_K = 2
_BLK = 128
_NF = 2
_GW = 32
_CW = 16
_LANES = 16


def _sc_mesh():
    return plsc.VectorSubcoreMesh(core_axis_name="core", subcore_axis_name="subcore")


_NW = 32


def _sc_gather(xf, idx):
    (P,) = idx.shape
    _, D = xf.shape
    per_w = P // _NW
    n_ch = per_w // _GW

    @functools.partial(
        pl.kernel,
        out_type=jax.ShapeDtypeStruct((P, D), xf.dtype),
        mesh=_sc_mesh(),
        scratch_types=[
            pltpu.VMEM((_GW,), jnp.int32),
            pltpu.VMEM((_GW, D), xf.dtype),
            pltpu.SemaphoreType.DMA,
        ],
    )
    def gk(x_hbm, i_hbm, o_hbm, idx_v, rows_v, sem):
        wid = lax.axis_index("core") * 16 + lax.axis_index("subcore")
        base = wid * per_w

        @pl.loop(0, n_ch)
        def _(c):
            off = base + c * _GW
            pltpu.sync_copy(i_hbm.at[pl.ds(off, _GW)], idx_v)
            pltpu.async_copy(x_hbm.at[idx_v], rows_v, sem).wait()
            pltpu.sync_copy(rows_v, o_hbm.at[pl.ds(off, _GW)])

    return gk(xf, idx)


def _sc_combine(y, i0, i1):
    _, D = y.shape
    T = i0.shape[0]
    per_w = T // _NW
    n_ch = per_w // _CW

    @functools.partial(
        pl.kernel,
        out_type=jax.ShapeDtypeStruct((T, D), y.dtype),
        mesh=_sc_mesh(),
        scratch_types=[
            pltpu.VMEM((_CW,), jnp.int32),
            pltpu.VMEM((_CW,), jnp.int32),
            pltpu.VMEM((_CW, D), y.dtype),
            pltpu.VMEM((_CW, D), y.dtype),
            pltpu.SemaphoreType.DMA,
        ],
    )
    def ck(y_hbm, i0_hbm, i1_hbm, o_hbm, i0_v, i1_v, b0, b1, sem):
        wid = lax.axis_index("core") * 16 + lax.axis_index("subcore")
        base = wid * per_w

        @pl.loop(0, n_ch)
        def _(c):
            off = base + c * _CW
            pltpu.sync_copy(i0_hbm.at[pl.ds(off, _CW)], i0_v)
            pltpu.sync_copy(i1_hbm.at[pl.ds(off, _CW)], i1_v)
            pltpu.async_copy(y_hbm.at[i0_v], b0, sem).wait()
            pltpu.async_copy(y_hbm.at[i1_v], b1, sem).wait()

            @pl.loop(0, _CW)
            def _(r):
                @pl.loop(0, D, step=_LANES)
                def _(cc):
                    slc = (pl.ds(r, 1), pl.ds(cc, _LANES))
                    b0.at[slc][...] = b0.at[slc][...] + b1.at[slc][...]

            pltpu.sync_copy(b0, o_hbm.at[pl.ds(off, _CW)])

    return ck(y, i0, i1)


def _tc_grouped_ffn(rows, scores, block_expert, W1, W3, W2):
    P, D = rows.shape
    E, _, FF = W1.shape
    NP = P // _BLK
    FC = FF // _NF
    scores2 = scores.reshape(P, 1)

    def fk(be_ref, xs_ref, sc_ref, w1_ref, w3_ref, w2_ref, o_ref):
        xsc = xs_ref[...] * sc_ref[...]
        u = jnp.dot(xsc, w1_ref[0], preferred_element_type=jnp.float32)
        v = jnp.dot(xsc, w3_ref[0], preferred_element_type=jnp.float32)
        h = (u / (1.0 + jnp.exp(-u))) * v
        y = jnp.dot(h, w2_ref[0], preferred_element_type=jnp.float32)
        c = pl.program_id(1)

        @pl.when(c == 0)
        def _():
            o_ref[...] = y

        @pl.when(c != 0)
        def _():
            o_ref[...] += y

    grid_spec = pltpu.PrefetchScalarGridSpec(
        num_scalar_prefetch=1,
        grid=(NP, _NF),
        in_specs=[
            pl.BlockSpec((_BLK, D), lambda i, c, be: (i, 0)),
            pl.BlockSpec((_BLK, 1), lambda i, c, be: (i, 0)),
            pl.BlockSpec((1, D, FC), lambda i, c, be: (be[i], 0, c)),
            pl.BlockSpec((1, D, FC), lambda i, c, be: (be[i], 0, c)),
            pl.BlockSpec((1, FC, D), lambda i, c, be: (be[i], c, 0)),
        ],
        out_specs=pl.BlockSpec((_BLK, D), lambda i, c, be: (i, 0)),
    )
    return pl.pallas_call(
        fk,
        grid_spec=grid_spec,
        out_shape=jax.ShapeDtypeStruct((P, D), rows.dtype),
    )(block_expert, rows, scores2, W1, W3, W2)


def kernel(x, Wr, W1, W2, W3):
    bs, slen, dim = x.shape
    T = bs * slen
    E = Wr.shape[1]
    xf = x.reshape(T, dim)

    logits = xf @ Wr
    probs = jax.nn.softmax(logits, axis=-1)
    top_s, top_i = jax.lax.top_k(probs, _TOP_K)
    expert_ids = top_i.reshape(-1).astype(jnp.int32)
    scores = top_s.reshape(-1)
    A = T * _TOP_K

    onehot = (expert_ids[:, None] == jnp.arange(E, dtype=jnp.int32)[None, :]).astype(jnp.int32)
    csum = jnp.cumsum(onehot, axis=0)
    counts = csum[-1]
    rank = jnp.take_along_axis(csum, expert_ids[:, None], axis=1)[:, 0] - 1

    padded_counts = ((counts + _BLK - 1) // _BLK) * _BLK
    poffs = jnp.concatenate(
        [jnp.zeros((1,), jnp.int32), jnp.cumsum(padded_counts)[:-1].astype(jnp.int32)]
    )
    dest = poffs[expert_ids] + rank
    NPAD = A + E * _BLK
    tok = (jnp.arange(A, dtype=jnp.int32) // _TOP_K)
    tok_pad = jnp.zeros((NPAD,), jnp.int32).at[dest].set(tok)
    score_pad = jnp.zeros((NPAD,), scores.dtype).at[dest].set(scores)

    NP = NPAD // _BLK
    blk_starts = jnp.arange(NP, dtype=jnp.int32) * _BLK
    block_expert = jnp.clip(
        jnp.searchsorted(poffs, blk_starts, side="right").astype(jnp.int32) - 1, 0, E - 1
    )

    routed = _sc_gather(xf, tok_pad)
    y = _tc_grouped_ffn(routed, score_pad, block_expert, W1, W3, W2)
    inv = dest.reshape(T, _TOP_K)
    out = _sc_combine(y, inv[:, 0], inv[:, 1])
    return out.reshape(bs, slen, dim)


# speedup vs baseline: 1.8587x; 1.2064x over previous
import dataclasses
import functools

import jax
import jax.numpy as jnp
from jax import lax
from jax.experimental import pallas as pl
from jax.experimental.pallas import tpu as pltpu
from jax.experimental.pallas import tpu_sc as plsc

_TOP_K = 2
_BLK = 128
_NF = 2
_CW = 16
_LANES = 16
_NW = 32
_SB = 256


def _sc_mesh():
    return plsc.VectorSubcoreMesh(core_axis_name="core", subcore_axis_name="subcore")


def _sc_params():
    cp = pltpu.CompilerParams()
    if "needs_layout_passes" in pltpu.CompilerParams.__dataclass_fields__:
        cp = dataclasses.replace(cp, needs_layout_passes=False)
    return cp


def _tc_router(xf, Wr):
    T, D = xf.shape
    E = Wr.shape[1]
    NPAD = T * _TOP_K + E * _BLK
    NP = NPAD // _BLK
    NB = T // _SB

    def rk(xf_ref, wr_ref, d0_ref, d1_ref, s0_ref, s1_ref, be_ref, ps_ref):
        logits = jnp.dot(xf_ref[...], wr_ref[...], preferred_element_type=jnp.float32)
        lmax = jnp.max(logits, axis=1, keepdims=True)
        el = jnp.exp(logits - lmax)
        probs = el / jnp.sum(el, axis=1, keepdims=True)

        lane = lax.broadcasted_iota(jnp.int32, (T, E), 1)
        m0 = jnp.max(probs, axis=1, keepdims=True)
        i0 = jnp.min(jnp.where(probs == m0, lane, E), axis=1, keepdims=True)
        masked = jnp.where(lane == i0, -jnp.inf, probs)
        m1 = jnp.max(masked, axis=1, keepdims=True)
        i1 = jnp.min(jnp.where(masked == m1, lane, E), axis=1, keepdims=True)

        oh = (jnp.where(lane == i0, 1.0, 0.0) + jnp.where(lane == i1, 1.0, 0.0))

        r_s = lax.broadcasted_iota(jnp.int32, (_SB, _SB), 0)
        c_s = lax.broadcasted_iota(jnp.int32, (_SB, _SB), 1)
        Ls = jnp.where(c_s < r_s, 1.0, 0.0)
        r_b = lax.broadcasted_iota(jnp.int32, (NB, NB), 0)
        c_b = lax.broadcasted_iota(jnp.int32, (NB, NB), 1)
        Lb = jnp.where(c_b < r_b, 1.0, 0.0)

        pcs = []
        tots = []
        for b in range(NB):
            ohb = oh[b * _SB:(b + 1) * _SB, :]
            pcs.append(jnp.dot(Ls, ohb, preferred_element_type=jnp.float32))
            tots.append(jnp.sum(ohb, axis=0, keepdims=True))
        tot = jnp.concatenate(tots, axis=0)
        bpre = jnp.dot(Lb, tot, preferred_element_type=jnp.float32)
        prevcount = jnp.concatenate(
            [pcs[b] + bpre[b:b + 1, :] for b in range(NB)], axis=0
        )

        counts = jnp.sum(tot, axis=0, keepdims=True)
        padded = jnp.floor((counts + (_BLK - 1)) / _BLK) * _BLK
        e_row = lax.broadcasted_iota(jnp.int32, (E, E), 0)
        e_col = lax.broadcasted_iota(jnp.int32, (E, E), 1)
        Ue = jnp.where(e_row < e_col, 1.0, 0.0)
        poffs = jnp.dot(padded, Ue, preferred_element_type=jnp.float32)

        rank0 = jnp.sum(jnp.where(lane == i0, prevcount, 0.0), axis=1)
        rank1 = jnp.sum(jnp.where(lane == i1, prevcount, 0.0), axis=1)
        off0 = jnp.sum(jnp.where(lane == i0, poffs, 0.0), axis=1)
        off1 = jnp.sum(jnp.where(lane == i1, poffs, 0.0), axis=1)
        d0_ref[...] = (off0 + rank0).astype(jnp.int32)
        d1_ref[...] = (off1 + rank1).astype(jnp.int32)
        s0_ref[...] = jnp.sum(jnp.where(lane == i0, probs, 0.0), axis=1)
        s1_ref[...] = jnp.sum(jnp.where(lane == i1, probs, 0.0), axis=1)

        blk0 = lax.broadcasted_iota(jnp.int32, (NP, E), 0) * _BLK
        e_lane = lax.broadcasted_iota(jnp.int32, (NP, E), 1)
        pof = jnp.broadcast_to(poffs, (NP, E))
        pad = jnp.broadcast_to(padded, (NP, E))
        cnt = jnp.broadcast_to(counts, (NP, E))
        blk0f = blk0.astype(jnp.float32)
        in_range = jnp.where((pof <= blk0f) & (blk0f < pof + pad), 1.0, 0.0)
        be_ref[...] = jnp.sum(e_lane.astype(jnp.float32) * in_range, axis=1).astype(jnp.int32)
        pad_end = jnp.sum((pof + cnt) * in_range, axis=1)
        ps_ref[...] = pad_end.astype(jnp.int32) - lax.iota(jnp.int32, NP) * _BLK

    return pl.pallas_call(
        rk,
        out_shape=(
            jax.ShapeDtypeStruct((T,), jnp.int32),
            jax.ShapeDtypeStruct((T,), jnp.int32),
            jax.ShapeDtypeStruct((T,), jnp.float32),
            jax.ShapeDtypeStruct((T,), jnp.float32),
            jax.ShapeDtypeStruct((NP,), jnp.int32),
            jax.ShapeDtypeStruct((NP,), jnp.int32),
        ),
    )(xf, Wr)


def _sc_dispatch_build(T, D, E):
    NPAD = T * _TOP_K + E * _BLK
    per_w = T // _NW

    @functools.partial(
        pl.kernel,
        out_type=(
            jax.ShapeDtypeStruct((NPAD, D), jnp.float32),
            jax.ShapeDtypeStruct((NPAD,), jnp.float32),
        ),
        mesh=_sc_mesh(),
        compiler_params=_sc_params(),
        scratch_types=[
            pltpu.VMEM((per_w, D), jnp.float32),
            pltpu.VMEM((per_w,), jnp.int32),
            pltpu.VMEM((per_w,), jnp.int32),
            pltpu.VMEM((T,), jnp.int32),
            pltpu.VMEM((T,), jnp.int32),
            pltpu.VMEM((T,), jnp.float32),
            pltpu.VMEM((T,), jnp.float32),
            pltpu.VMEM((NPAD,), jnp.float32),
            pltpu.SemaphoreType.DMA,
        ],
    )
    def bk(xf_hbm, d0_hbm, d1_hbm, s0_hbm, s1_hbm, routed_hbm, spad_hbm,
           rows_v, d0_v, d1_v, ad0_v, ad1_v, as0_v, as1_v, spad_v, sem):
        wid = lax.axis_index("core") * 16 + lax.axis_index("subcore")
        tb = wid * per_w
        pltpu.sync_copy(d0_hbm.at[pl.ds(tb, per_w)], d0_v)
        pltpu.sync_copy(d1_hbm.at[pl.ds(tb, per_w)], d1_v)
        pltpu.sync_copy(xf_hbm.at[pl.ds(tb, per_w)], rows_v)
        pltpu.sync_copy(rows_v, routed_hbm.at[d0_v])
        pltpu.sync_copy(rows_v, routed_hbm.at[d1_v])

        @pl.when(wid == 0)
        def _():
            @pl.loop(0, NPAD, step=_LANES)
            def _(i):
                spad_v[pl.ds(i, _LANES)] = jnp.zeros((_LANES,), jnp.float32)

            pltpu.sync_copy(d0_hbm, ad0_v)
            pltpu.sync_copy(d1_hbm, ad1_v)
            pltpu.sync_copy(s0_hbm, as0_v)
            pltpu.sync_copy(s1_hbm, as1_v)

            @pl.loop(0, T, step=_LANES)
            def _(i):
                plsc.store_scatter(spad_v, [ad0_v[pl.ds(i, _LANES)]],
                                   as0_v[pl.ds(i, _LANES)])
                plsc.store_scatter(spad_v, [ad1_v[pl.ds(i, _LANES)]],
                                   as1_v[pl.ds(i, _LANES)])

            pltpu.sync_copy(spad_v, spad_hbm)

    return bk


def _tc_grouped_ffn(rows, spad, block_expert, pad_start, W1, W3, W2):
    P, D = rows.shape
    E, _, FF = W1.shape
    NP = P // _BLK
    FC = FF // _NF
    spad2 = spad.reshape(P, 1)

    def fk(be_ref, ps_ref, xs_ref, sc_ref, w1_ref, w3_ref, w2_ref, o_ref):
        i = pl.program_id(0)
        row = lax.broadcasted_iota(jnp.int32, (_BLK, 1), 0)
        valid = row < ps_ref[i]
        xsc = jnp.where(valid, xs_ref[...] * sc_ref[...], 0.0)
        u = jnp.dot(xsc, w1_ref[0], preferred_element_type=jnp.float32)
        v = jnp.dot(xsc, w3_ref[0], preferred_element_type=jnp.float32)
        h = (u / (1.0 + jnp.exp(-u))) * v
        y = jnp.dot(h, w2_ref[0], preferred_element_type=jnp.float32)
        c = pl.program_id(1)

        @pl.when(c == 0)
        def _():
            o_ref[...] = y

        @pl.when(c != 0)
        def _():
            o_ref[...] += y

    grid_spec = pltpu.PrefetchScalarGridSpec(
        num_scalar_prefetch=2,
        grid=(NP, _NF),
        in_specs=[
            pl.BlockSpec((_BLK, D), lambda i, c, be, ps: (i, 0)),
            pl.BlockSpec((_BLK, 1), lambda i, c, be, ps: (i, 0)),
            pl.BlockSpec((1, D, FC), lambda i, c, be, ps: (be[i], 0, c)),
            pl.BlockSpec((1, D, FC), lambda i, c, be, ps: (be[i], 0, c)),
            pl.BlockSpec((1, FC, D), lambda i, c, be, ps: (be[i], c, 0)),
        ],
        out_specs=pl.BlockSpec((_BLK, D), lambda i, c, be, ps: (i, 0)),
    )
    return pl.pallas_call(
        fk,
        grid_spec=grid_spec,
        out_shape=jax.ShapeDtypeStruct((P, D), rows.dtype),
    )(block_expert, pad_start, rows, spad2, W1, W3, W2)


def _sc_combine(y, i0, i1):
    _, D = y.shape
    T = i0.shape[0]
    per_w = T // _NW
    n_ch = per_w // _CW

    @functools.partial(
        pl.kernel,
        out_type=jax.ShapeDtypeStruct((T, D), y.dtype),
        mesh=_sc_mesh(),
        scratch_types=[
            pltpu.VMEM((_CW,), jnp.int32),
            pltpu.VMEM((_CW,), jnp.int32),
            pltpu.VMEM((_CW, D), y.dtype),
            pltpu.VMEM((_CW, D), y.dtype),
            pltpu.SemaphoreType.DMA,
        ],
    )
    def ck(y_hbm, i0_hbm, i1_hbm, o_hbm, i0_v, i1_v, b0, b1, sem):
        wid = lax.axis_index("core") * 16 + lax.axis_index("subcore")
        base = wid * per_w

        @pl.loop(0, n_ch)
        def _(c):
            off = base + c * _CW
            pltpu.sync_copy(i0_hbm.at[pl.ds(off, _CW)], i0_v)
            pltpu.sync_copy(i1_hbm.at[pl.ds(off, _CW)], i1_v)
            pltpu.async_copy(y_hbm.at[i0_v], b0, sem).wait()
            pltpu.async_copy(y_hbm.at[i1_v], b1, sem).wait()

            @pl.loop(0, _CW)
            def _(r):
                @pl.loop(0, D, step=_LANES)
                def _(cc):
                    slc = (pl.ds(r, 1), pl.ds(cc, _LANES))
                    b0.at[slc][...] = b0.at[slc][...] + b1.at[slc][...]

            pltpu.sync_copy(b0, o_hbm.at[pl.ds(off, _CW)])

    return ck(y, i0, i1)


def kernel(x, Wr, W1, W2, W3):
    bs, slen, dim = x.shape
    T = bs * slen
    E = Wr.shape[1]
    xf = x.reshape(T, dim)

    d0, d1, s0, s1, block_expert, pad_start = _tc_router(xf, Wr)
    routed, spad = _sc_dispatch_build(T, dim, E)(xf, d0, d1, s0, s1)
    y = _tc_grouped_ffn(routed, spad, block_expert, pad_start, W1, W3, W2)
    out = _sc_combine(y, d0, d1)
    return out.reshape(bs, slen, dim)


# speedup vs baseline: 3.1709x; 1.7060x over previous
import dataclasses
import functools

import jax
import jax.numpy as jnp
from jax import lax
from jax.experimental import pallas as pl
from jax.experimental.pallas import tpu as pltpu
from jax.experimental.pallas import tpu_sc as plsc

_TOP_K = 2
_BLK = 128
_NF = 1
_CW = 16
_LANES = 16
_NW = 32
_SB = 256


def _sc_mesh():
    return plsc.VectorSubcoreMesh(core_axis_name="core", subcore_axis_name="subcore")


def _sc_params():
    cp = pltpu.CompilerParams()
    if "needs_layout_passes" in pltpu.CompilerParams.__dataclass_fields__:
        cp = dataclasses.replace(cp, needs_layout_passes=False)
    return cp


def _tc_router(xf, Wr):
    T, D = xf.shape
    E = Wr.shape[1]
    NPAD = T * _TOP_K + E * _BLK
    NP = NPAD // _BLK
    NB = T // _SB

    def rk(xf_ref, wr_ref, d0_ref, d1_ref, s0_ref, s1_ref, be_ref, ps_ref):
        logits = jnp.dot(xf_ref[...], wr_ref[...], preferred_element_type=jnp.float32)
        lmax = jnp.max(logits, axis=1, keepdims=True)
        el = jnp.exp(logits - lmax)
        probs = el / jnp.sum(el, axis=1, keepdims=True)

        lane = lax.broadcasted_iota(jnp.int32, (T, E), 1)
        m0 = jnp.max(probs, axis=1, keepdims=True)
        i0 = jnp.min(jnp.where(probs == m0, lane, E), axis=1, keepdims=True)
        masked = jnp.where(lane == i0, -jnp.inf, probs)
        m1 = jnp.max(masked, axis=1, keepdims=True)
        i1 = jnp.min(jnp.where(masked == m1, lane, E), axis=1, keepdims=True)

        oh = (jnp.where(lane == i0, 1.0, 0.0) + jnp.where(lane == i1, 1.0, 0.0))

        r_s = lax.broadcasted_iota(jnp.int32, (_SB, _SB), 0)
        c_s = lax.broadcasted_iota(jnp.int32, (_SB, _SB), 1)
        Ls = jnp.where(c_s < r_s, 1.0, 0.0)
        r_b = lax.broadcasted_iota(jnp.int32, (NB, NB), 0)
        c_b = lax.broadcasted_iota(jnp.int32, (NB, NB), 1)
        Lb = jnp.where(c_b < r_b, 1.0, 0.0)

        pcs = []
        tots = []
        for b in range(NB):
            ohb = oh[b * _SB:(b + 1) * _SB, :]
            pcs.append(jnp.dot(Ls, ohb, preferred_element_type=jnp.float32))
            tots.append(jnp.sum(ohb, axis=0, keepdims=True))
        tot = jnp.concatenate(tots, axis=0)
        bpre = jnp.dot(Lb, tot, preferred_element_type=jnp.float32)
        prevcount = jnp.concatenate(
            [pcs[b] + bpre[b:b + 1, :] for b in range(NB)], axis=0
        )

        counts = jnp.sum(tot, axis=0, keepdims=True)
        padded = jnp.floor((counts + (_BLK - 1)) / _BLK) * _BLK
        e_row = lax.broadcasted_iota(jnp.int32, (E, E), 0)
        e_col = lax.broadcasted_iota(jnp.int32, (E, E), 1)
        Ue = jnp.where(e_row < e_col, 1.0, 0.0)
        poffs = jnp.dot(padded, Ue, preferred_element_type=jnp.float32)

        rank0 = jnp.sum(jnp.where(lane == i0, prevcount, 0.0), axis=1)
        rank1 = jnp.sum(jnp.where(lane == i1, prevcount, 0.0), axis=1)
        off0 = jnp.sum(jnp.where(lane == i0, poffs, 0.0), axis=1)
        off1 = jnp.sum(jnp.where(lane == i1, poffs, 0.0), axis=1)
        d0_ref[...] = (off0 + rank0).astype(jnp.int32)
        d1_ref[...] = (off1 + rank1).astype(jnp.int32)
        s0_ref[...] = jnp.sum(jnp.where(lane == i0, probs, 0.0), axis=1)
        s1_ref[...] = jnp.sum(jnp.where(lane == i1, probs, 0.0), axis=1)

        blk0 = lax.broadcasted_iota(jnp.int32, (NP, E), 0) * _BLK
        e_lane = lax.broadcasted_iota(jnp.int32, (NP, E), 1)
        pof = jnp.broadcast_to(poffs, (NP, E))
        pad = jnp.broadcast_to(padded, (NP, E))
        cnt = jnp.broadcast_to(counts, (NP, E))
        blk0f = blk0.astype(jnp.float32)
        in_range = jnp.where((pof <= blk0f) & (blk0f < pof + pad), 1.0, 0.0)
        be_ref[...] = jnp.sum(e_lane.astype(jnp.float32) * in_range, axis=1).astype(jnp.int32)
        pad_end = jnp.sum((pof + cnt) * in_range, axis=1)
        ps_ref[...] = pad_end.astype(jnp.int32) - lax.iota(jnp.int32, NP) * _BLK

    return pl.pallas_call(
        rk,
        out_shape=(
            jax.ShapeDtypeStruct((T,), jnp.int32),
            jax.ShapeDtypeStruct((T,), jnp.int32),
            jax.ShapeDtypeStruct((T,), jnp.float32),
            jax.ShapeDtypeStruct((T,), jnp.float32),
            jax.ShapeDtypeStruct((NP,), jnp.int32),
            jax.ShapeDtypeStruct((NP,), jnp.int32),
        ),
    )(xf, Wr)


def _sc_dispatch_build(T, D, E):
    NPAD = T * _TOP_K + E * _BLK
    per_w = T // _NW

    @functools.partial(
        pl.kernel,
        out_type=(
            jax.ShapeDtypeStruct((NPAD, D), jnp.float32),
            jax.ShapeDtypeStruct((NPAD,), jnp.float32),
        ),
        mesh=_sc_mesh(),
        compiler_params=_sc_params(),
        scratch_types=[
            pltpu.VMEM((per_w, D), jnp.float32),
            pltpu.VMEM((per_w,), jnp.int32),
            pltpu.VMEM((per_w,), jnp.int32),
            pltpu.VMEM((T,), jnp.int32),
            pltpu.VMEM((T,), jnp.int32),
            pltpu.VMEM((T,), jnp.float32),
            pltpu.VMEM((T,), jnp.float32),
            pltpu.VMEM((NPAD,), jnp.float32),
            pltpu.SemaphoreType.DMA,
        ],
    )
    def bk(xf_hbm, d0_hbm, d1_hbm, s0_hbm, s1_hbm, routed_hbm, spad_hbm,
           rows_v, d0_v, d1_v, ad0_v, ad1_v, as0_v, as1_v, spad_v, sem):
        wid = lax.axis_index("core") * 16 + lax.axis_index("subcore")
        tb = wid * per_w
        pltpu.sync_copy(d0_hbm.at[pl.ds(tb, per_w)], d0_v)
        pltpu.sync_copy(d1_hbm.at[pl.ds(tb, per_w)], d1_v)
        pltpu.sync_copy(xf_hbm.at[pl.ds(tb, per_w)], rows_v)
        pltpu.sync_copy(rows_v, routed_hbm.at[d0_v])
        pltpu.sync_copy(rows_v, routed_hbm.at[d1_v])

        @pl.when(wid == 0)
        def _():
            @pl.loop(0, NPAD, step=_LANES)
            def _(i):
                spad_v[pl.ds(i, _LANES)] = jnp.zeros((_LANES,), jnp.float32)

            pltpu.sync_copy(d0_hbm, ad0_v)
            pltpu.sync_copy(d1_hbm, ad1_v)
            pltpu.sync_copy(s0_hbm, as0_v)
            pltpu.sync_copy(s1_hbm, as1_v)

            @pl.loop(0, T, step=_LANES)
            def _(i):
                plsc.store_scatter(spad_v, [ad0_v[pl.ds(i, _LANES)]],
                                   as0_v[pl.ds(i, _LANES)])
                plsc.store_scatter(spad_v, [ad1_v[pl.ds(i, _LANES)]],
                                   as1_v[pl.ds(i, _LANES)])

            pltpu.sync_copy(spad_v, spad_hbm)

    return bk


def _tc_grouped_ffn(rows, spad, block_expert, pad_start, W1, W3, W2):
    P, D = rows.shape
    E, _, FF = W1.shape
    NP = P // _BLK
    FC = FF // _NF
    spad2 = spad.reshape(P, 1)

    def fk(be_ref, ps_ref, xs_ref, sc_ref, w1_ref, w3_ref, w2_ref, o_ref):
        i = pl.program_id(0)
        row = lax.broadcasted_iota(jnp.int32, (_BLK, 1), 0)
        valid = row < ps_ref[i]
        xsc = jnp.where(valid, xs_ref[...] * sc_ref[...], 0.0)
        u = jnp.dot(xsc, w1_ref[0], preferred_element_type=jnp.float32)
        v = jnp.dot(xsc, w3_ref[0], preferred_element_type=jnp.float32)
        h = (u / (1.0 + jnp.exp(-u))) * v
        y = jnp.dot(h, w2_ref[0], preferred_element_type=jnp.float32)
        c = pl.program_id(1)

        @pl.when(c == 0)
        def _():
            o_ref[...] = y

        @pl.when(c != 0)
        def _():
            o_ref[...] += y

    grid_spec = pltpu.PrefetchScalarGridSpec(
        num_scalar_prefetch=2,
        grid=(NP, _NF),
        in_specs=[
            pl.BlockSpec((_BLK, D), lambda i, c, be, ps: (i, 0)),
            pl.BlockSpec((_BLK, 1), lambda i, c, be, ps: (i, 0)),
            pl.BlockSpec((1, D, FC), lambda i, c, be, ps: (be[i], 0, c)),
            pl.BlockSpec((1, D, FC), lambda i, c, be, ps: (be[i], 0, c)),
            pl.BlockSpec((1, FC, D), lambda i, c, be, ps: (be[i], c, 0)),
        ],
        out_specs=pl.BlockSpec((_BLK, D), lambda i, c, be, ps: (i, 0)),
    )
    return pl.pallas_call(
        fk,
        grid_spec=grid_spec,
        out_shape=jax.ShapeDtypeStruct((P, D), rows.dtype),
    )(block_expert, pad_start, rows, spad2, W1, W3, W2)


def _sc_combine(y, i0, i1):
    _, D = y.shape
    T = i0.shape[0]
    per_w = T // _NW
    n_ch = per_w // _CW

    @functools.partial(
        pl.kernel,
        out_type=jax.ShapeDtypeStruct((T, D), y.dtype),
        mesh=_sc_mesh(),
        scratch_types=[
            pltpu.VMEM((_CW,), jnp.int32),
            pltpu.VMEM((_CW,), jnp.int32),
            pltpu.VMEM((_CW, D), y.dtype),
            pltpu.VMEM((_CW, D), y.dtype),
            pltpu.SemaphoreType.DMA,
        ],
    )
    def ck(y_hbm, i0_hbm, i1_hbm, o_hbm, i0_v, i1_v, b0, b1, sem):
        wid = lax.axis_index("core") * 16 + lax.axis_index("subcore")
        base = wid * per_w

        @pl.loop(0, n_ch)
        def _(c):
            off = base + c * _CW
            pltpu.sync_copy(i0_hbm.at[pl.ds(off, _CW)], i0_v)
            pltpu.sync_copy(i1_hbm.at[pl.ds(off, _CW)], i1_v)
            pltpu.async_copy(y_hbm.at[i0_v], b0, sem).wait()
            pltpu.async_copy(y_hbm.at[i1_v], b1, sem).wait()

            @pl.loop(0, _CW)
            def _(r):
                @pl.loop(0, D, step=_LANES)
                def _(cc):
                    slc = (pl.ds(r, 1), pl.ds(cc, _LANES))
                    b0.at[slc][...] = b0.at[slc][...] + b1.at[slc][...]

            pltpu.sync_copy(b0, o_hbm.at[pl.ds(off, _CW)])

    return ck(y, i0, i1)


def kernel(x, Wr, W1, W2, W3):
    bs, slen, dim = x.shape
    T = bs * slen
    E = Wr.shape[1]
    xf = x.reshape(T, dim)

    d0, d1, s0, s1, block_expert, pad_start = _tc_router(xf, Wr)
    routed, spad = _sc_dispatch_build(T, dim, E)(xf, d0, d1, s0, s1)
    y = _tc_grouped_ffn(routed, spad, block_expert, pad_start, W1, W3, W2)
    out = _sc_combine(y, d0, d1)
    return out.reshape(bs, slen, dim)


# speedup vs baseline: 3.1712x; 1.0001x over previous
import dataclasses
import functools

import jax
import jax.numpy as jnp
from jax import lax
from jax.experimental import pallas as pl
from jax.experimental.pallas import tpu as pltpu
from jax.experimental.pallas import tpu_sc as plsc

_TOP_K = 2
_BLK = 128
_NF = 1
_CW = 16
_LANES = 16
_NW = 32
_SB = 256


def _sc_mesh():
    return plsc.VectorSubcoreMesh(core_axis_name="core", subcore_axis_name="subcore")


def _sc_params():
    cp = pltpu.CompilerParams()
    if "needs_layout_passes" in pltpu.CompilerParams.__dataclass_fields__:
        cp = dataclasses.replace(cp, needs_layout_passes=False)
    return cp


def _tc_router(xf, Wr):
    T, D = xf.shape
    E = Wr.shape[1]
    NPAD = T * _TOP_K + E * _BLK
    NP = NPAD // _BLK
    NB = T // _SB

    def rk(xf_ref, wr_ref, d0_ref, d1_ref, s0_ref, s1_ref, be_ref, ps_ref):
        logits = jnp.dot(xf_ref[...], wr_ref[...], preferred_element_type=jnp.float32)
        lmax = jnp.max(logits, axis=1, keepdims=True)
        el = jnp.exp(logits - lmax)
        probs = el / jnp.sum(el, axis=1, keepdims=True)

        lane = lax.broadcasted_iota(jnp.int32, (T, E), 1)
        m0 = jnp.max(probs, axis=1, keepdims=True)
        i0 = jnp.min(jnp.where(probs == m0, lane, E), axis=1, keepdims=True)
        masked = jnp.where(lane == i0, -jnp.inf, probs)
        m1 = jnp.max(masked, axis=1, keepdims=True)
        i1 = jnp.min(jnp.where(masked == m1, lane, E), axis=1, keepdims=True)

        oh = (jnp.where(lane == i0, 1.0, 0.0) + jnp.where(lane == i1, 1.0, 0.0))

        r_s = lax.broadcasted_iota(jnp.int32, (_SB, _SB), 0)
        c_s = lax.broadcasted_iota(jnp.int32, (_SB, _SB), 1)
        Ls = jnp.where(c_s < r_s, 1.0, 0.0)
        r_b = lax.broadcasted_iota(jnp.int32, (NB, NB), 0)
        c_b = lax.broadcasted_iota(jnp.int32, (NB, NB), 1)
        Lb = jnp.where(c_b < r_b, 1.0, 0.0)

        pcs = []
        tots = []
        for b in range(NB):
            ohb = oh[b * _SB:(b + 1) * _SB, :]
            pcs.append(jnp.dot(Ls, ohb, preferred_element_type=jnp.float32))
            tots.append(jnp.sum(ohb, axis=0, keepdims=True))
        tot = jnp.concatenate(tots, axis=0)
        bpre = jnp.dot(Lb, tot, preferred_element_type=jnp.float32)
        prevcount = jnp.concatenate(
            [pcs[b] + bpre[b:b + 1, :] for b in range(NB)], axis=0
        )

        counts = jnp.sum(tot, axis=0, keepdims=True)
        padded = jnp.floor((counts + (_BLK - 1)) / _BLK) * _BLK
        e_row = lax.broadcasted_iota(jnp.int32, (E, E), 0)
        e_col = lax.broadcasted_iota(jnp.int32, (E, E), 1)
        Ue = jnp.where(e_row < e_col, 1.0, 0.0)
        poffs = jnp.dot(padded, Ue, preferred_element_type=jnp.float32)

        rank0 = jnp.sum(jnp.where(lane == i0, prevcount, 0.0), axis=1)
        rank1 = jnp.sum(jnp.where(lane == i1, prevcount, 0.0), axis=1)
        off0 = jnp.sum(jnp.where(lane == i0, poffs, 0.0), axis=1)
        off1 = jnp.sum(jnp.where(lane == i1, poffs, 0.0), axis=1)
        d0_ref[...] = (off0 + rank0).astype(jnp.int32)
        d1_ref[...] = (off1 + rank1).astype(jnp.int32)
        s0_ref[...] = jnp.sum(jnp.where(lane == i0, probs, 0.0), axis=1)
        s1_ref[...] = jnp.sum(jnp.where(lane == i1, probs, 0.0), axis=1)

        blk0 = lax.broadcasted_iota(jnp.int32, (NP, E), 0) * _BLK
        e_lane = lax.broadcasted_iota(jnp.int32, (NP, E), 1)
        pof = jnp.broadcast_to(poffs, (NP, E))
        pad = jnp.broadcast_to(padded, (NP, E))
        cnt = jnp.broadcast_to(counts, (NP, E))
        blk0f = blk0.astype(jnp.float32)
        in_range = jnp.where((pof <= blk0f) & (blk0f < pof + pad), 1.0, 0.0)
        be_ref[...] = jnp.sum(e_lane.astype(jnp.float32) * in_range, axis=1).astype(jnp.int32)
        pad_end = jnp.sum((pof + cnt) * in_range, axis=1)
        ps_ref[...] = pad_end.astype(jnp.int32) - lax.iota(jnp.int32, NP) * _BLK

    return pl.pallas_call(
        rk,
        out_shape=(
            jax.ShapeDtypeStruct((T,), jnp.int32),
            jax.ShapeDtypeStruct((T,), jnp.int32),
            jax.ShapeDtypeStruct((T,), jnp.float32),
            jax.ShapeDtypeStruct((T,), jnp.float32),
            jax.ShapeDtypeStruct((NP,), jnp.int32),
            jax.ShapeDtypeStruct((NP,), jnp.int32),
        ),
    )(xf, Wr)


def _sc_dispatch_build(T, D, E):
    NPAD = T * _TOP_K + E * _BLK
    per_w = T // _NW

    @functools.partial(
        pl.kernel,
        out_type=(
            jax.ShapeDtypeStruct((NPAD, D), jnp.float32),
            jax.ShapeDtypeStruct((NPAD,), jnp.float32),
        ),
        mesh=_sc_mesh(),
        compiler_params=_sc_params(),
        scratch_types=[
            pltpu.VMEM((per_w, D), jnp.float32),
            pltpu.VMEM((per_w,), jnp.int32),
            pltpu.VMEM((per_w,), jnp.int32),
            pltpu.VMEM((T,), jnp.int32),
            pltpu.VMEM((T,), jnp.int32),
            pltpu.VMEM((T,), jnp.float32),
            pltpu.VMEM((T,), jnp.float32),
            pltpu.VMEM((NPAD,), jnp.float32),
            pltpu.SemaphoreType.DMA,
        ],
    )
    def bk(xf_hbm, d0_hbm, d1_hbm, s0_hbm, s1_hbm, routed_hbm, spad_hbm,
           rows_v, d0_v, d1_v, ad0_v, ad1_v, as0_v, as1_v, spad_v, sem):
        wid = lax.axis_index("core") * 16 + lax.axis_index("subcore")
        tb = wid * per_w
        pltpu.sync_copy(d0_hbm.at[pl.ds(tb, per_w)], d0_v)
        pltpu.sync_copy(d1_hbm.at[pl.ds(tb, per_w)], d1_v)
        pltpu.sync_copy(xf_hbm.at[pl.ds(tb, per_w)], rows_v)
        pltpu.sync_copy(rows_v, routed_hbm.at[d0_v])
        pltpu.sync_copy(rows_v, routed_hbm.at[d1_v])

        @pl.when(wid == 0)
        def _():
            @pl.loop(0, NPAD, step=_LANES)
            def _(i):
                spad_v[pl.ds(i, _LANES)] = jnp.zeros((_LANES,), jnp.float32)

            pltpu.sync_copy(d0_hbm, ad0_v)
            pltpu.sync_copy(d1_hbm, ad1_v)
            pltpu.sync_copy(s0_hbm, as0_v)
            pltpu.sync_copy(s1_hbm, as1_v)

            @pl.loop(0, T, step=_LANES)
            def _(i):
                plsc.store_scatter(spad_v, [ad0_v[pl.ds(i, _LANES)]],
                                   as0_v[pl.ds(i, _LANES)])
                plsc.store_scatter(spad_v, [ad1_v[pl.ds(i, _LANES)]],
                                   as1_v[pl.ds(i, _LANES)])

            pltpu.sync_copy(spad_v, spad_hbm)

    return bk


def _tc_grouped_ffn(rows, spad, block_expert, pad_start, W1, W3, W2):
    P, D = rows.shape
    E, _, FF = W1.shape
    NP = P // _BLK
    FC = FF // _NF
    spad2 = spad.reshape(P, 1)

    def fk(be_ref, ps_ref, xs_ref, sc_ref, w1_ref, w3_ref, w2_ref, o_ref):
        i = pl.program_id(0)
        row = lax.broadcasted_iota(jnp.int32, (_BLK, 1), 0)
        valid = row < ps_ref[i]
        xsc = jnp.where(valid, xs_ref[...] * sc_ref[...], 0.0)
        xb = xsc.astype(jnp.bfloat16)
        u = jnp.dot(xb, w1_ref[0].astype(jnp.bfloat16), preferred_element_type=jnp.float32)
        v = jnp.dot(xb, w3_ref[0].astype(jnp.bfloat16), preferred_element_type=jnp.float32)
        h = (u / (1.0 + jnp.exp(-u))) * v
        y = jnp.dot(h.astype(jnp.bfloat16), w2_ref[0].astype(jnp.bfloat16),
                    preferred_element_type=jnp.float32)
        c = pl.program_id(1)

        @pl.when(c == 0)
        def _():
            o_ref[...] = y

        @pl.when(c != 0)
        def _():
            o_ref[...] += y

    grid_spec = pltpu.PrefetchScalarGridSpec(
        num_scalar_prefetch=2,
        grid=(NP, _NF),
        in_specs=[
            pl.BlockSpec((_BLK, D), lambda i, c, be, ps: (i, 0)),
            pl.BlockSpec((_BLK, 1), lambda i, c, be, ps: (i, 0)),
            pl.BlockSpec((1, D, FC), lambda i, c, be, ps: (be[i], 0, c)),
            pl.BlockSpec((1, D, FC), lambda i, c, be, ps: (be[i], 0, c)),
            pl.BlockSpec((1, FC, D), lambda i, c, be, ps: (be[i], c, 0)),
        ],
        out_specs=pl.BlockSpec((_BLK, D), lambda i, c, be, ps: (i, 0)),
    )
    return pl.pallas_call(
        fk,
        grid_spec=grid_spec,
        out_shape=jax.ShapeDtypeStruct((P, D), rows.dtype),
    )(block_expert, pad_start, rows, spad2, W1, W3, W2)


def _sc_combine(y, i0, i1):
    _, D = y.shape
    T = i0.shape[0]
    per_w = T // _NW
    n_ch = per_w // _CW

    @functools.partial(
        pl.kernel,
        out_type=jax.ShapeDtypeStruct((T, D), y.dtype),
        mesh=_sc_mesh(),
        scratch_types=[
            pltpu.VMEM((_CW,), jnp.int32),
            pltpu.VMEM((_CW,), jnp.int32),
            pltpu.VMEM((_CW, D), y.dtype),
            pltpu.VMEM((_CW, D), y.dtype),
            pltpu.SemaphoreType.DMA,
        ],
    )
    def ck(y_hbm, i0_hbm, i1_hbm, o_hbm, i0_v, i1_v, b0, b1, sem):
        wid = lax.axis_index("core") * 16 + lax.axis_index("subcore")
        base = wid * per_w

        @pl.loop(0, n_ch)
        def _(c):
            off = base + c * _CW
            pltpu.sync_copy(i0_hbm.at[pl.ds(off, _CW)], i0_v)
            pltpu.sync_copy(i1_hbm.at[pl.ds(off, _CW)], i1_v)
            pltpu.async_copy(y_hbm.at[i0_v], b0, sem).wait()
            pltpu.async_copy(y_hbm.at[i1_v], b1, sem).wait()

            @pl.loop(0, _CW)
            def _(r):
                @pl.loop(0, D, step=_LANES)
                def _(cc):
                    slc = (pl.ds(r, 1), pl.ds(cc, _LANES))
                    b0.at[slc][...] = b0.at[slc][...] + b1.at[slc][...]

            pltpu.sync_copy(b0, o_hbm.at[pl.ds(off, _CW)])

    return ck(y, i0, i1)


def kernel(x, Wr, W1, W2, W3):
    bs, slen, dim = x.shape
    T = bs * slen
    E = Wr.shape[1]
    xf = x.reshape(T, dim)

    d0, d1, s0, s1, block_expert, pad_start = _tc_router(xf, Wr)
    routed, spad = _sc_dispatch_build(T, dim, E)(xf, d0, d1, s0, s1)
    y = _tc_grouped_ffn(routed, spad, block_expert, pad_start, W1, W3, W2)
    out = _sc_combine(y, d0, d1)
    return out.reshape(bs, slen, dim)


# speedup vs baseline: 3.4512x; 1.0883x over previous
import dataclasses
import functools

import jax
import jax.numpy as jnp
from jax import lax
from jax.experimental import pallas as pl
from jax.experimental.pallas import tpu as pltpu
from jax.experimental.pallas import tpu_sc as plsc

_TOP_K = 2
_BLK = 128
_NF = 1
_CW = 16
_LANES = 16
_NW = 32
_SB = 256


def _sc_mesh():
    return plsc.VectorSubcoreMesh(core_axis_name="core", subcore_axis_name="subcore")


def _sc_params():
    cp = pltpu.CompilerParams()
    if "needs_layout_passes" in pltpu.CompilerParams.__dataclass_fields__:
        cp = dataclasses.replace(cp, needs_layout_passes=False)
    return cp


def _tc_router(xf, Wr):
    T, D = xf.shape
    E = Wr.shape[1]
    NPAD = T * _TOP_K + E * _BLK
    NP = NPAD // _BLK
    NB = T // _SB

    def rk(xf_ref, wr_ref, d0_ref, d1_ref, s0_ref, s1_ref, be_ref, ps_ref,
           chg_ref, slot_ref, nxt_ref, hasn_ref):
        logits = jnp.dot(xf_ref[...], wr_ref[...], preferred_element_type=jnp.float32)
        lmax = jnp.max(logits, axis=1, keepdims=True)
        el = jnp.exp(logits - lmax)
        probs = el / jnp.sum(el, axis=1, keepdims=True)

        lane = lax.broadcasted_iota(jnp.int32, (T, E), 1)
        m0 = jnp.max(probs, axis=1, keepdims=True)
        i0 = jnp.min(jnp.where(probs == m0, lane, E), axis=1, keepdims=True)
        masked = jnp.where(lane == i0, -jnp.inf, probs)
        m1 = jnp.max(masked, axis=1, keepdims=True)
        i1 = jnp.min(jnp.where(masked == m1, lane, E), axis=1, keepdims=True)

        oh = (jnp.where(lane == i0, 1.0, 0.0) + jnp.where(lane == i1, 1.0, 0.0))

        r_s = lax.broadcasted_iota(jnp.int32, (_SB, _SB), 0)
        c_s = lax.broadcasted_iota(jnp.int32, (_SB, _SB), 1)
        Ls = jnp.where(c_s < r_s, 1.0, 0.0)
        r_b = lax.broadcasted_iota(jnp.int32, (NB, NB), 0)
        c_b = lax.broadcasted_iota(jnp.int32, (NB, NB), 1)
        Lb = jnp.where(c_b < r_b, 1.0, 0.0)

        pcs = []
        tots = []
        for b in range(NB):
            ohb = oh[b * _SB:(b + 1) * _SB, :]
            pcs.append(jnp.dot(Ls, ohb, preferred_element_type=jnp.float32))
            tots.append(jnp.sum(ohb, axis=0, keepdims=True))
        tot = jnp.concatenate(tots, axis=0)
        bpre = jnp.dot(Lb, tot, preferred_element_type=jnp.float32)
        prevcount = jnp.concatenate(
            [pcs[b] + bpre[b:b + 1, :] for b in range(NB)], axis=0
        )

        counts = jnp.sum(tot, axis=0, keepdims=True)
        padded = jnp.floor((counts + (_BLK - 1)) / _BLK) * _BLK
        e_row = lax.broadcasted_iota(jnp.int32, (E, E), 0)
        e_col = lax.broadcasted_iota(jnp.int32, (E, E), 1)
        Ue = jnp.where(e_row < e_col, 1.0, 0.0)
        poffs = jnp.dot(padded, Ue, preferred_element_type=jnp.float32)

        rank0 = jnp.sum(jnp.where(lane == i0, prevcount, 0.0), axis=1)
        rank1 = jnp.sum(jnp.where(lane == i1, prevcount, 0.0), axis=1)
        off0 = jnp.sum(jnp.where(lane == i0, poffs, 0.0), axis=1)
        off1 = jnp.sum(jnp.where(lane == i1, poffs, 0.0), axis=1)
        d0_ref[...] = (off0 + rank0).astype(jnp.int32)
        d1_ref[...] = (off1 + rank1).astype(jnp.int32)
        s0_ref[...] = jnp.sum(jnp.where(lane == i0, probs, 0.0), axis=1)
        s1_ref[...] = jnp.sum(jnp.where(lane == i1, probs, 0.0), axis=1)

        blk0 = lax.broadcasted_iota(jnp.int32, (NP, E), 0) * _BLK
        e_lane = lax.broadcasted_iota(jnp.int32, (NP, E), 1)
        pof = jnp.broadcast_to(poffs, (NP, E))
        pad = jnp.broadcast_to(padded, (NP, E))
        cnt = jnp.broadcast_to(counts, (NP, E))
        blk0f = blk0.astype(jnp.float32)
        e_lane_f = e_lane.astype(jnp.float32)
        in_range = jnp.where((pof <= blk0f) & (blk0f < pof + pad), 1.0, 0.0)
        be_ref[...] = jnp.sum(e_lane_f * in_range, axis=1).astype(jnp.int32)
        pad_end = jnp.sum((pof + cnt) * in_range, axis=1)
        ps_ref[...] = pad_end.astype(jnp.int32) - lax.iota(jnp.int32, NP) * _BLK

        present = jnp.where(padded > 0.0, 1.0, 0.0)
        pres = jnp.broadcast_to(present, (NP, E))
        mystart = jnp.sum(pof * in_range, axis=1, keepdims=True)
        mynext = jnp.sum((pof + pad) * in_range, axis=1, keepdims=True)
        chg_ref[...] = jnp.sum(jnp.where(in_range * jnp.where(blk0f == pof, 1.0, 0.0) > 0.0,
                                         1.0, 0.0), axis=1).astype(jnp.int32)
        gcnt = jnp.sum(jnp.where((pres > 0.0) & (pof <= mystart), 1.0, 0.0), axis=1)
        slot_ref[...] = (gcnt.astype(jnp.int32) - 1) & 1
        nxt_ind = jnp.where((pres > 0.0) & (pof <= mynext) & (mynext < pof + pad), 1.0, 0.0)
        nxt_ref[...] = jnp.sum(e_lane_f * nxt_ind, axis=1).astype(jnp.int32)
        hasn_ref[...] = jnp.sum(nxt_ind, axis=1).astype(jnp.int32)

    return pl.pallas_call(
        rk,
        out_shape=(
            jax.ShapeDtypeStruct((T,), jnp.int32),
            jax.ShapeDtypeStruct((T,), jnp.int32),
            jax.ShapeDtypeStruct((T,), jnp.float32),
            jax.ShapeDtypeStruct((T,), jnp.float32),
            jax.ShapeDtypeStruct((NP,), jnp.int32),
            jax.ShapeDtypeStruct((NP,), jnp.int32),
            jax.ShapeDtypeStruct((NP,), jnp.int32),
            jax.ShapeDtypeStruct((NP,), jnp.int32),
            jax.ShapeDtypeStruct((NP,), jnp.int32),
            jax.ShapeDtypeStruct((NP,), jnp.int32),
        ),
    )(xf, Wr)


def _sc_dispatch_build(T, D, E):
    NPAD = T * _TOP_K + E * _BLK
    per_w = T // _NW

    @functools.partial(
        pl.kernel,
        out_type=(
            jax.ShapeDtypeStruct((NPAD, D), jnp.float32),
            jax.ShapeDtypeStruct((NPAD,), jnp.float32),
        ),
        mesh=_sc_mesh(),
        compiler_params=_sc_params(),
        scratch_types=[
            pltpu.VMEM((per_w, D), jnp.float32),
            pltpu.VMEM((per_w,), jnp.int32),
            pltpu.VMEM((per_w,), jnp.int32),
            pltpu.VMEM((T,), jnp.int32),
            pltpu.VMEM((T,), jnp.int32),
            pltpu.VMEM((T,), jnp.float32),
            pltpu.VMEM((T,), jnp.float32),
            pltpu.VMEM((NPAD,), jnp.float32),
            pltpu.SemaphoreType.DMA,
        ],
    )
    def bk(xf_hbm, d0_hbm, d1_hbm, s0_hbm, s1_hbm, routed_hbm, spad_hbm,
           rows_v, d0_v, d1_v, ad0_v, ad1_v, as0_v, as1_v, spad_v, sem):
        wid = lax.axis_index("core") * 16 + lax.axis_index("subcore")
        tb = wid * per_w
        pltpu.sync_copy(d0_hbm.at[pl.ds(tb, per_w)], d0_v)
        pltpu.sync_copy(d1_hbm.at[pl.ds(tb, per_w)], d1_v)
        pltpu.sync_copy(xf_hbm.at[pl.ds(tb, per_w)], rows_v)
        pltpu.sync_copy(rows_v, routed_hbm.at[d0_v])
        pltpu.sync_copy(rows_v, routed_hbm.at[d1_v])

        @pl.when(wid == 0)
        def _():
            @pl.loop(0, NPAD, step=_LANES)
            def _(i):
                spad_v[pl.ds(i, _LANES)] = jnp.zeros((_LANES,), jnp.float32)

            pltpu.sync_copy(d0_hbm, ad0_v)
            pltpu.sync_copy(d1_hbm, ad1_v)
            pltpu.sync_copy(s0_hbm, as0_v)
            pltpu.sync_copy(s1_hbm, as1_v)

            @pl.loop(0, T, step=_LANES)
            def _(i):
                plsc.store_scatter(spad_v, [ad0_v[pl.ds(i, _LANES)]],
                                   as0_v[pl.ds(i, _LANES)])
                plsc.store_scatter(spad_v, [ad1_v[pl.ds(i, _LANES)]],
                                   as1_v[pl.ds(i, _LANES)])

            pltpu.sync_copy(spad_v, spad_hbm)

    return bk


def _tc_grouped_ffn(rows, spad, be, ps, chg, slot, nxt, hasn, W1, W3, W2):
    P, D = rows.shape
    E, _, FF = W1.shape
    NP = P // _BLK
    spad2 = spad.reshape(P, 1)

    def fk(be_ref, ps_ref, chg_ref, slot_ref, nxt_ref, hasn_ref,
           xs_ref, sc_ref, w1_hbm, w3_hbm, w2_hbm, o_ref,
           w1s, w3s, w2s, sem1, sem3, sem2):
        i = pl.program_id(0)

        def start(eidx, sl):
            pltpu.make_async_copy(w1_hbm.at[eidx], w1s.at[sl], sem1.at[sl]).start()
            pltpu.make_async_copy(w3_hbm.at[eidx], w3s.at[sl], sem3.at[sl]).start()
            pltpu.make_async_copy(w2_hbm.at[eidx], w2s.at[sl], sem2.at[sl]).start()

        @pl.when(i == 0)
        def _():
            start(be_ref[0], slot_ref[0])

        @pl.when(chg_ref[i] == 1)
        def _():
            @pl.when(hasn_ref[i] == 1)
            def _():
                start(nxt_ref[i], 1 - slot_ref[i])

            sl = slot_ref[i]
            e = be_ref[i]
            pltpu.make_async_copy(w1_hbm.at[e], w1s.at[sl], sem1.at[sl]).wait()
            pltpu.make_async_copy(w3_hbm.at[e], w3s.at[sl], sem3.at[sl]).wait()
            pltpu.make_async_copy(w2_hbm.at[e], w2s.at[sl], sem2.at[sl]).wait()

        row = lax.broadcasted_iota(jnp.int32, (_BLK, 1), 0)
        valid = row < ps_ref[i]
        xsc = jnp.where(valid, xs_ref[...] * sc_ref[...], 0.0)
        xb = xsc.astype(jnp.bfloat16)
        sl = slot_ref[i]
        u = jnp.dot(xb, w1s[sl].astype(jnp.bfloat16), preferred_element_type=jnp.float32)
        v = jnp.dot(xb, w3s[sl].astype(jnp.bfloat16), preferred_element_type=jnp.float32)
        h = (u / (1.0 + jnp.exp(-u))) * v
        o_ref[...] = jnp.dot(h.astype(jnp.bfloat16), w2s[sl].astype(jnp.bfloat16),
                             preferred_element_type=jnp.float32)

    grid_spec = pltpu.PrefetchScalarGridSpec(
        num_scalar_prefetch=6,
        grid=(NP,),
        in_specs=[
            pl.BlockSpec((_BLK, D), lambda i, *_: (i, 0)),
            pl.BlockSpec((_BLK, 1), lambda i, *_: (i, 0)),
            pl.BlockSpec(memory_space=pl.ANY),
            pl.BlockSpec(memory_space=pl.ANY),
            pl.BlockSpec(memory_space=pl.ANY),
        ],
        out_specs=pl.BlockSpec((_BLK, D), lambda i, *_: (i, 0)),
        scratch_shapes=[
            pltpu.VMEM((2, D, FF), jnp.float32),
            pltpu.VMEM((2, D, FF), jnp.float32),
            pltpu.VMEM((2, FF, D), jnp.float32),
            pltpu.SemaphoreType.DMA((2,)),
            pltpu.SemaphoreType.DMA((2,)),
            pltpu.SemaphoreType.DMA((2,)),
        ],
    )
    return pl.pallas_call(
        fk,
        grid_spec=grid_spec,
        out_shape=jax.ShapeDtypeStruct((P, D), rows.dtype),
    )(be, ps, chg, slot, nxt, hasn, rows, spad2, W1, W3, W2)


def _sc_combine(y, i0, i1):
    _, D = y.shape
    T = i0.shape[0]
    per_w = T // _NW
    n_ch = per_w // _CW

    @functools.partial(
        pl.kernel,
        out_type=jax.ShapeDtypeStruct((T, D), y.dtype),
        mesh=_sc_mesh(),
        scratch_types=[
            pltpu.VMEM((_CW,), jnp.int32),
            pltpu.VMEM((_CW,), jnp.int32),
            pltpu.VMEM((_CW, D), y.dtype),
            pltpu.VMEM((_CW, D), y.dtype),
            pltpu.SemaphoreType.DMA,
        ],
    )
    def ck(y_hbm, i0_hbm, i1_hbm, o_hbm, i0_v, i1_v, b0, b1, sem):
        wid = lax.axis_index("core") * 16 + lax.axis_index("subcore")
        base = wid * per_w

        @pl.loop(0, n_ch)
        def _(c):
            off = base + c * _CW
            pltpu.sync_copy(i0_hbm.at[pl.ds(off, _CW)], i0_v)
            pltpu.sync_copy(i1_hbm.at[pl.ds(off, _CW)], i1_v)
            pltpu.async_copy(y_hbm.at[i0_v], b0, sem).wait()
            pltpu.async_copy(y_hbm.at[i1_v], b1, sem).wait()

            @pl.loop(0, _CW)
            def _(r):
                @pl.loop(0, D, step=_LANES)
                def _(cc):
                    slc = (pl.ds(r, 1), pl.ds(cc, _LANES))
                    b0.at[slc][...] = b0.at[slc][...] + b1.at[slc][...]

            pltpu.sync_copy(b0, o_hbm.at[pl.ds(off, _CW)])

    return ck(y, i0, i1)


def kernel(x, Wr, W1, W2, W3):
    bs, slen, dim = x.shape
    T = bs * slen
    E = Wr.shape[1]
    xf = x.reshape(T, dim)

    d0, d1, s0, s1, be, ps, chg, slot, nxt, hasn = _tc_router(xf, Wr)
    routed, spad = _sc_dispatch_build(T, dim, E)(xf, d0, d1, s0, s1)
    y = _tc_grouped_ffn(routed, spad, be, ps, chg, slot, nxt, hasn, W1, W3, W2)
    out = _sc_combine(y, d0, d1)
    return out.reshape(bs, slen, dim)


# speedup vs baseline: 3.5852x; 1.0388x over previous
import dataclasses
import functools

import jax
import jax.numpy as jnp
from jax import lax
from jax.experimental import pallas as pl
from jax.experimental.pallas import tpu as pltpu
from jax.experimental.pallas import tpu_sc as plsc

_TOP_K = 2
_BLK = 128
_NF = 1
_CW = 16
_LANES = 16
_NW = 32
_SB = 256


def _sc_mesh():
    return plsc.VectorSubcoreMesh(core_axis_name="core", subcore_axis_name="subcore")


def _sc_params():
    cp = pltpu.CompilerParams()
    if "needs_layout_passes" in pltpu.CompilerParams.__dataclass_fields__:
        cp = dataclasses.replace(cp, needs_layout_passes=False)
    return cp


def _tc_router(xf, Wr):
    T, D = xf.shape
    E = Wr.shape[1]
    NPAD = T * _TOP_K + E * _BLK
    NP = NPAD // _BLK
    NB = T // _SB

    def rk(xf_ref, wr_ref, d0_ref, d1_ref, s0_ref, s1_ref, be_ref, ps_ref,
           chg_ref, slot_ref, nxt_ref, hasn_ref):
        logits = jnp.dot(xf_ref[...], wr_ref[...], preferred_element_type=jnp.float32)
        lmax = jnp.max(logits, axis=1, keepdims=True)
        el = jnp.exp(logits - lmax)
        probs = el / jnp.sum(el, axis=1, keepdims=True)

        lane = lax.broadcasted_iota(jnp.int32, (T, E), 1)
        m0 = jnp.max(probs, axis=1, keepdims=True)
        i0 = jnp.min(jnp.where(probs == m0, lane, E), axis=1, keepdims=True)
        masked = jnp.where(lane == i0, -jnp.inf, probs)
        m1 = jnp.max(masked, axis=1, keepdims=True)
        i1 = jnp.min(jnp.where(masked == m1, lane, E), axis=1, keepdims=True)

        oh = (jnp.where(lane == i0, 1.0, 0.0) + jnp.where(lane == i1, 1.0, 0.0))

        r_s = lax.broadcasted_iota(jnp.int32, (_SB, _SB), 0)
        c_s = lax.broadcasted_iota(jnp.int32, (_SB, _SB), 1)
        Ls = jnp.where(c_s < r_s, 1.0, 0.0)
        r_b = lax.broadcasted_iota(jnp.int32, (NB, NB), 0)
        c_b = lax.broadcasted_iota(jnp.int32, (NB, NB), 1)
        Lb = jnp.where(c_b < r_b, 1.0, 0.0)

        pcs = []
        tots = []
        for b in range(NB):
            ohb = oh[b * _SB:(b + 1) * _SB, :]
            pcs.append(jnp.dot(Ls, ohb, preferred_element_type=jnp.float32))
            tots.append(jnp.sum(ohb, axis=0, keepdims=True))
        tot = jnp.concatenate(tots, axis=0)
        bpre = jnp.dot(Lb, tot, preferred_element_type=jnp.float32)
        prevcount = jnp.concatenate(
            [pcs[b] + bpre[b:b + 1, :] for b in range(NB)], axis=0
        )

        counts = jnp.sum(tot, axis=0, keepdims=True)
        padded = jnp.floor((counts + (_BLK - 1)) / _BLK) * _BLK
        e_row = lax.broadcasted_iota(jnp.int32, (E, E), 0)
        e_col = lax.broadcasted_iota(jnp.int32, (E, E), 1)
        Ue = jnp.where(e_row < e_col, 1.0, 0.0)
        poffs = jnp.dot(padded, Ue, preferred_element_type=jnp.float32)

        rank0 = jnp.sum(jnp.where(lane == i0, prevcount, 0.0), axis=1)
        rank1 = jnp.sum(jnp.where(lane == i1, prevcount, 0.0), axis=1)
        off0 = jnp.sum(jnp.where(lane == i0, poffs, 0.0), axis=1)
        off1 = jnp.sum(jnp.where(lane == i1, poffs, 0.0), axis=1)
        d0_ref[...] = (off0 + rank0).astype(jnp.int32)
        d1_ref[...] = (off1 + rank1).astype(jnp.int32)
        s0_ref[...] = jnp.sum(jnp.where(lane == i0, probs, 0.0), axis=1)
        s1_ref[...] = jnp.sum(jnp.where(lane == i1, probs, 0.0), axis=1)

        blk0 = lax.broadcasted_iota(jnp.int32, (NP, E), 0) * _BLK
        e_lane = lax.broadcasted_iota(jnp.int32, (NP, E), 1)
        pof = jnp.broadcast_to(poffs, (NP, E))
        pad = jnp.broadcast_to(padded, (NP, E))
        cnt = jnp.broadcast_to(counts, (NP, E))
        blk0f = blk0.astype(jnp.float32)
        e_lane_f = e_lane.astype(jnp.float32)
        in_range = jnp.where((pof <= blk0f) & (blk0f < pof + pad), 1.0, 0.0)
        be_ref[...] = jnp.sum(e_lane_f * in_range, axis=1).astype(jnp.int32)
        pad_end = jnp.sum((pof + cnt) * in_range, axis=1)
        ps_ref[...] = pad_end.astype(jnp.int32) - lax.iota(jnp.int32, NP) * _BLK

        present = jnp.where(padded > 0.0, 1.0, 0.0)
        pres = jnp.broadcast_to(present, (NP, E))
        mystart = jnp.sum(pof * in_range, axis=1, keepdims=True)
        mynext = jnp.sum((pof + pad) * in_range, axis=1, keepdims=True)
        chg_ref[...] = jnp.sum(jnp.where(in_range * jnp.where(blk0f == pof, 1.0, 0.0) > 0.0,
                                         1.0, 0.0), axis=1).astype(jnp.int32)
        gcnt = jnp.sum(jnp.where((pres > 0.0) & (pof <= mystart), 1.0, 0.0), axis=1)
        slot_ref[...] = (gcnt.astype(jnp.int32) - 1) & 1
        nxt_ind = jnp.where((pres > 0.0) & (pof <= mynext) & (mynext < pof + pad), 1.0, 0.0)
        nxt_ref[...] = jnp.sum(e_lane_f * nxt_ind, axis=1).astype(jnp.int32)
        hasn_ref[...] = jnp.sum(nxt_ind, axis=1).astype(jnp.int32)

    return pl.pallas_call(
        rk,
        out_shape=(
            jax.ShapeDtypeStruct((T,), jnp.int32),
            jax.ShapeDtypeStruct((T,), jnp.int32),
            jax.ShapeDtypeStruct((T,), jnp.float32),
            jax.ShapeDtypeStruct((T,), jnp.float32),
            jax.ShapeDtypeStruct((NP,), jnp.int32),
            jax.ShapeDtypeStruct((NP,), jnp.int32),
            jax.ShapeDtypeStruct((NP,), jnp.int32),
            jax.ShapeDtypeStruct((NP,), jnp.int32),
            jax.ShapeDtypeStruct((NP,), jnp.int32),
            jax.ShapeDtypeStruct((NP,), jnp.int32),
        ),
    )(xf, Wr)


def _sc_dispatch_build(T, D, E):
    NPAD = T * _TOP_K + E * _BLK
    per_w = T // _NW

    @functools.partial(
        pl.kernel,
        out_type=(
            jax.ShapeDtypeStruct((NPAD, D), jnp.float32),
            jax.ShapeDtypeStruct((NPAD,), jnp.float32),
        ),
        mesh=_sc_mesh(),
        compiler_params=_sc_params(),
        scratch_types=[
            pltpu.VMEM((per_w, D), jnp.float32),
            pltpu.VMEM((per_w,), jnp.int32),
            pltpu.VMEM((per_w,), jnp.int32),
            pltpu.VMEM((T,), jnp.int32),
            pltpu.VMEM((T,), jnp.int32),
            pltpu.VMEM((T,), jnp.float32),
            pltpu.VMEM((T,), jnp.float32),
            pltpu.VMEM((NPAD,), jnp.float32),
            pltpu.SemaphoreType.DMA,
        ],
    )
    def bk(xf_hbm, d0_hbm, d1_hbm, s0_hbm, s1_hbm, routed_hbm, spad_hbm,
           rows_v, d0_v, d1_v, ad0_v, ad1_v, as0_v, as1_v, spad_v, sem):
        wid = lax.axis_index("core") * 16 + lax.axis_index("subcore")
        tb = wid * per_w
        pltpu.sync_copy(d0_hbm.at[pl.ds(tb, per_w)], d0_v)
        pltpu.sync_copy(d1_hbm.at[pl.ds(tb, per_w)], d1_v)
        pltpu.sync_copy(xf_hbm.at[pl.ds(tb, per_w)], rows_v)
        c0 = pltpu.make_async_copy(rows_v, routed_hbm.at[d0_v], sem)
        c1 = pltpu.make_async_copy(rows_v, routed_hbm.at[d1_v], sem)
        c0.start()
        c1.start()
        c0.wait()
        c1.wait()

        @pl.when(wid == 0)
        def _():
            @pl.loop(0, NPAD, step=_LANES)
            def _(i):
                spad_v[pl.ds(i, _LANES)] = jnp.zeros((_LANES,), jnp.float32)

            pltpu.sync_copy(d0_hbm, ad0_v)
            pltpu.sync_copy(d1_hbm, ad1_v)
            pltpu.sync_copy(s0_hbm, as0_v)
            pltpu.sync_copy(s1_hbm, as1_v)

            @pl.loop(0, T, step=_LANES)
            def _(i):
                plsc.store_scatter(spad_v, [ad0_v[pl.ds(i, _LANES)]],
                                   as0_v[pl.ds(i, _LANES)])
                plsc.store_scatter(spad_v, [ad1_v[pl.ds(i, _LANES)]],
                                   as1_v[pl.ds(i, _LANES)])

            pltpu.sync_copy(spad_v, spad_hbm)

    return bk


def _tc_grouped_ffn(rows, spad, be, ps, chg, slot, nxt, hasn, W1, W3, W2):
    P, D = rows.shape
    E, _, FF = W1.shape
    NP = P // _BLK
    spad2 = spad.reshape(P, 1)

    def fk(be_ref, ps_ref, chg_ref, slot_ref, nxt_ref, hasn_ref,
           xs_ref, sc_ref, w1_hbm, w3_hbm, w2_hbm, o_ref,
           w1s, w3s, w2s, sem1, sem3, sem2):
        i = pl.program_id(0)

        def start(eidx, sl):
            pltpu.make_async_copy(w1_hbm.at[eidx], w1s.at[sl], sem1.at[sl]).start()
            pltpu.make_async_copy(w3_hbm.at[eidx], w3s.at[sl], sem3.at[sl]).start()
            pltpu.make_async_copy(w2_hbm.at[eidx], w2s.at[sl], sem2.at[sl]).start()

        @pl.when(i == 0)
        def _():
            start(be_ref[0], slot_ref[0])

        @pl.when(chg_ref[i] == 1)
        def _():
            @pl.when(hasn_ref[i] == 1)
            def _():
                start(nxt_ref[i], 1 - slot_ref[i])

            sl = slot_ref[i]
            e = be_ref[i]
            pltpu.make_async_copy(w1_hbm.at[e], w1s.at[sl], sem1.at[sl]).wait()
            pltpu.make_async_copy(w3_hbm.at[e], w3s.at[sl], sem3.at[sl]).wait()
            pltpu.make_async_copy(w2_hbm.at[e], w2s.at[sl], sem2.at[sl]).wait()

        row = lax.broadcasted_iota(jnp.int32, (_BLK, 1), 0)
        valid = row < ps_ref[i]
        xsc = jnp.where(valid, xs_ref[...] * sc_ref[...], 0.0)
        xb = xsc.astype(jnp.bfloat16)
        sl = slot_ref[i]
        u = jnp.dot(xb, w1s[sl].astype(jnp.bfloat16), preferred_element_type=jnp.float32)
        v = jnp.dot(xb, w3s[sl].astype(jnp.bfloat16), preferred_element_type=jnp.float32)
        h = (u / (1.0 + jnp.exp(-u))) * v
        o_ref[...] = jnp.dot(h.astype(jnp.bfloat16), w2s[sl].astype(jnp.bfloat16),
                             preferred_element_type=jnp.float32)

    grid_spec = pltpu.PrefetchScalarGridSpec(
        num_scalar_prefetch=6,
        grid=(NP,),
        in_specs=[
            pl.BlockSpec((_BLK, D), lambda i, *_: (i, 0)),
            pl.BlockSpec((_BLK, 1), lambda i, *_: (i, 0)),
            pl.BlockSpec(memory_space=pl.ANY),
            pl.BlockSpec(memory_space=pl.ANY),
            pl.BlockSpec(memory_space=pl.ANY),
        ],
        out_specs=pl.BlockSpec((_BLK, D), lambda i, *_: (i, 0)),
        scratch_shapes=[
            pltpu.VMEM((2, D, FF), jnp.float32),
            pltpu.VMEM((2, D, FF), jnp.float32),
            pltpu.VMEM((2, FF, D), jnp.float32),
            pltpu.SemaphoreType.DMA((2,)),
            pltpu.SemaphoreType.DMA((2,)),
            pltpu.SemaphoreType.DMA((2,)),
        ],
    )
    return pl.pallas_call(
        fk,
        grid_spec=grid_spec,
        out_shape=jax.ShapeDtypeStruct((P, D), rows.dtype),
    )(be, ps, chg, slot, nxt, hasn, rows, spad2, W1, W3, W2)


def _sc_combine(y, i0, i1):
    _, D = y.shape
    T = i0.shape[0]
    per_w = T // _NW
    n_ch = per_w // _CW

    @functools.partial(
        pl.kernel,
        out_type=jax.ShapeDtypeStruct((T, D), y.dtype),
        mesh=_sc_mesh(),
        compiler_params=_sc_params(),
        scratch_types=[
            pltpu.VMEM((per_w,), jnp.int32),
            pltpu.VMEM((per_w,), jnp.int32),
            pltpu.VMEM((2, _CW, D), y.dtype),
            pltpu.VMEM((2, _CW, D), y.dtype),
            pltpu.SemaphoreType.DMA((2,)),
            pltpu.SemaphoreType.DMA((2,)),
        ],
    )
    def ck(y_hbm, i0_hbm, i1_hbm, o_hbm, i0_v, i1_v, b0, b1, gsem, osem):
        wid = lax.axis_index("core") * 16 + lax.axis_index("subcore")
        base = wid * per_w
        pltpu.sync_copy(i0_hbm.at[pl.ds(base, per_w)], i0_v)
        pltpu.sync_copy(i1_hbm.at[pl.ds(base, per_w)], i1_v)

        def gather(c, p):
            return (
                pltpu.make_async_copy(y_hbm.at[i0_v.at[pl.ds(c * _CW, _CW)]],
                                      b0.at[p], gsem.at[p]),
                pltpu.make_async_copy(y_hbm.at[i1_v.at[pl.ds(c * _CW, _CW)]],
                                      b1.at[p], gsem.at[p]),
            )

        def writeback(c, p):
            return pltpu.make_async_copy(
                b0.at[p], o_hbm.at[pl.ds(base + c * _CW, _CW)], osem.at[p])

        for g in gather(0, 0):
            g.start()
        for c in range(n_ch):
            p = c & 1
            if c + 1 < n_ch:
                if c >= 1:
                    writeback(c - 1, 1 - p).wait()
                for g in gather(c + 1, 1 - p):
                    g.start()
            for g in gather(c, p):
                g.wait()

            @pl.loop(0, _CW)
            def _(r):
                @pl.loop(0, D, step=_LANES, unroll=8)
                def _(cc):
                    slc = (p, r, pl.ds(cc, _LANES))
                    b0.at[slc][...] = b0.at[slc][...] + b1.at[slc][...]

            writeback(c, p).start()
        writeback(n_ch - 2, n_ch & 1).wait()
        writeback(n_ch - 1, (n_ch - 1) & 1).wait()

    return ck(y, i0, i1)


def kernel(x, Wr, W1, W2, W3):
    bs, slen, dim = x.shape
    T = bs * slen
    E = Wr.shape[1]
    xf = x.reshape(T, dim)

    d0, d1, s0, s1, be, ps, chg, slot, nxt, hasn = _tc_router(xf, Wr)
    routed, spad = _sc_dispatch_build(T, dim, E)(xf, d0, d1, s0, s1)
    y = _tc_grouped_ffn(routed, spad, be, ps, chg, slot, nxt, hasn, W1, W3, W2)
    out = _sc_combine(y, d0, d1)
    return out.reshape(bs, slen, dim)
